# Initial kernel scaffold; baseline (speedup 1.0000x reference)
import jax
import jax.numpy as jnp
from jax.experimental import pallas as pl


def kernel(x, memory_snapshot, delta_W, delta_b, xproj_W, xproj_b, phys_W, phys_b, gate_W, gate_b, outp_W, outp_b, seq_W, seq_b, q_W, q_b, mem_W, mem_b, curd_W, curd_b, memd_W, memd_b, fuse_W, fuse_b):
    raise NotImplementedError("write your pallas kernel here")


# speedup vs baseline: 4.1078x; 4.1078x over previous
import math

import jax
import jax.numpy as jnp
import numpy as np
from jax.experimental import pallas as pl
from jax.experimental.pallas import tpu as pltpu

_LB = 256
_LAMBDA_DRIFT = 0.3


def _pass1_body(x_ref, snap_ref, wd_ref, wx_ref, wp_ref, g12_ref, g31_ref,
                wo_ref, bd_ref, bu_ref, bg_ref, bo_ref,
                raw_ref, ms_ref, qd_ref):
    li = pl.program_id(1)
    T = snap_ref.shape[1]
    snap = snap_ref[0]
    xb = x_ref[0]
    xp = snap[T - 1]
    delta = xb - xp
    f32 = jnp.float32
    df = jnp.dot(delta, wd_ref[...], preferred_element_type=f32) + bd_ref[...]
    u = (jnp.dot(xb, wx_ref[...], preferred_element_type=f32)
         - jnp.dot(xp, wp_ref[...], preferred_element_type=f32) + bu_ref[...])
    g = jax.nn.sigmoid(jnp.dot(xb, g12_ref[...], preferred_element_type=f32)
                       + jnp.dot(xp, g31_ref[...], preferred_element_type=f32)
                       + bg_ref[...])
    core = g * df + (1.0 - g) * u
    raw = jnp.dot(core, wo_ref[...], preferred_element_type=f32) + bo_ref[...]
    raw_ref[0] = raw

    msp = jnp.sum(snap, axis=1)
    qp = jnp.sum(xb + raw, axis=0, keepdims=True)
    dp = jnp.sum(delta, axis=0, keepdims=True)
    qdp = jnp.concatenate([qp, dp], axis=0)

    @pl.when(li == 0)
    def _():
        ms_ref[0] = msp
        qd_ref[0] = qdp

    @pl.when(li > 0)
    def _():
        ms_ref[0] += msp
        qd_ref[0] += qdp


def _scores_body(ms_ref, qd_ref, pe_ref, seqw_ref, seqb_ref, qw_ref, qb_ref,
                 memw_ref, memb_ref, curdw_ref, curdb_ref, memdw_ref,
                 memdb_ref, f2_ref, fuseb_ref, attn_ref, pw_ref):
    B, T, D = ms_ref.shape
    L = 1024.0
    f32 = jnp.float32
    ms_mean = ms_ref[...] * (1.0 / L)
    pep = jnp.dot(pe_ref[...], seqw_ref[...],
                  preferred_element_type=f32) + seqb_ref[...]
    m = ms_mean + pep[None]
    memg = (jnp.dot(m.reshape(B * T, D), memw_ref[...],
                    preferred_element_type=f32)
            + memb_ref[...]).reshape(B, T, D)
    qg = jnp.dot(qd_ref[:, 0, :] * (1.0 / L), qw_ref[...],
                 preferred_element_type=f32) + qb_ref[...]
    content = jnp.sum(qg[:, None, :] * memg, axis=-1) / math.sqrt(D)
    cur = jnp.dot(qd_ref[:, 1, :] * (1.0 / L), curdw_ref[...],
                  preferred_element_type=f32) + curdb_ref[...]
    mprev = jnp.concatenate(
        [jnp.zeros((B, 1, D), f32), m[:, :-1]], axis=1)
    md = (m - mprev).reshape(B * T, D)
    memd = (jnp.dot(md, memdw_ref[...], preferred_element_type=f32)
            + memdb_ref[...]).reshape(B, T, D)
    drift = -jnp.mean((cur[:, None, :] - memd) ** 2, axis=-1)
    logits = content + _LAMBDA_DRIFT * drift
    mx = jnp.max(logits, axis=1, keepdims=True)
    e = jnp.exp(logits - mx)
    attn = e / jnp.sum(e, axis=1, keepdims=True)
    attn_ref[...] = attn
    pe_w = jnp.sum(attn[:, :, None] * pep[None], axis=1)
    pw2 = jnp.dot(pe_w, f2_ref[...],
                  preferred_element_type=f32) + fuseb_ref[...]
    pw_ref[...] = jnp.concatenate(
        [pe_w[:, None, :], pw2[:, None, :]], axis=1)


def _pass2_body(x_ref, snap_ref, raw_ref, attn_ref, pw_ref, f1_ref, f2_ref,
                y_ref):
    b = pl.program_id(0)
    T = snap_ref.shape[1]
    snap = snap_ref[0]
    xb = x_ref[0]
    raw = raw_ref[0]
    acc = snap[0] * attn_ref[b, 0]
    for t in range(1, T):
        acc = acc + snap[t] * attn_ref[b, t]
    pwv = pw_ref[0]
    enh = acc + pwv[0:1]
    f32 = jnp.float32
    fpre = (jnp.dot(xb, f1_ref[...], preferred_element_type=f32)
            + jnp.dot(acc, f2_ref[...], preferred_element_type=f32)
            + pwv[1:2])
    fg = jax.nn.sigmoid(fpre)
    y_ref[0] = xb + raw + fg * enh


def _sinusoid_np(T, d):
    half = d // 2
    pos = np.arange(1, T + 1, dtype=np.float64)
    div = np.exp(-math.log(10000.0) * (2.0 * np.arange(half) / d))
    ang = pos[:, None] * div[None, :]
    pe = np.stack([np.sin(ang), np.cos(ang)], axis=-1)
    return jnp.asarray(pe.reshape(T, d), dtype=jnp.float32)


def kernel(x, memory_snapshot, delta_W, delta_b, xproj_W, xproj_b, phys_W,
           phys_b, gate_W, gate_b, outp_W, outp_b, seq_W, seq_b, q_W, q_b,
           mem_W, mem_b, curd_W, curd_b, memd_W, memd_b, fuse_W, fuse_b):
    B, T, L, D = memory_snapshot.shape
    Lb = _LB
    nL = L // Lb

    g1 = gate_W[0:D]
    g12 = g1 + gate_W[D:2 * D]
    g31 = gate_W[2 * D:3 * D] - g1
    f1 = fuse_W[0:D]
    f2 = fuse_W[D:2 * D]
    bu = (xproj_b - phys_b).reshape(1, D)
    r1 = lambda v: v.reshape(1, D)

    wspec = pl.BlockSpec((D, D), lambda b_, l_: (0, 0))
    bspec = pl.BlockSpec((1, D), lambda b_, l_: (0, 0))
    raw, ms_sum, qd_sum = pl.pallas_call(
        _pass1_body,
        grid=(B, nL),
        in_specs=[
            pl.BlockSpec((1, Lb, D), lambda b_, l_: (b_, l_, 0)),
            pl.BlockSpec((1, T, Lb, D), lambda b_, l_: (b_, 0, l_, 0)),
            wspec, wspec, wspec, wspec, wspec, wspec,
            bspec, bspec, bspec, bspec,
        ],
        out_specs=[
            pl.BlockSpec((1, Lb, D), lambda b_, l_: (b_, l_, 0)),
            pl.BlockSpec((1, T, D), lambda b_, l_: (b_, 0, 0)),
            pl.BlockSpec((1, 2, D), lambda b_, l_: (b_, 0, 0)),
        ],
        out_shape=[
            jax.ShapeDtypeStruct((B, L, D), jnp.float32),
            jax.ShapeDtypeStruct((B, T, D), jnp.float32),
            jax.ShapeDtypeStruct((B, 2, D), jnp.float32),
        ],
        compiler_params=pltpu.CompilerParams(
            dimension_semantics=("parallel", "arbitrary"),
            vmem_limit_bytes=56 * 1024 * 1024,
        ),
        name="dalm_pass1",
    )(x, memory_snapshot, delta_W, xproj_W, phys_W, g12, g31, outp_W,
      r1(delta_b), bu, r1(gate_b), r1(outp_b))

    pe = _sinusoid_np(T, D)
    attn, pw = pl.pallas_call(
        _scores_body,
        out_shape=[
            jax.ShapeDtypeStruct((B, T), jnp.float32),
            jax.ShapeDtypeStruct((B, 2, D), jnp.float32),
        ],
        name="dalm_scores",
    )(ms_sum, qd_sum, pe, seq_W, r1(seq_b), q_W, r1(q_b), mem_W, r1(mem_b),
      curd_W, r1(curd_b), memd_W, r1(memd_b), f2, r1(fuse_b))

    y = pl.pallas_call(
        _pass2_body,
        grid=(B, nL),
        in_specs=[
            pl.BlockSpec((1, Lb, D), lambda b_, l_: (b_, l_, 0)),
            pl.BlockSpec((1, T, Lb, D), lambda b_, l_: (b_, 0, l_, 0)),
            pl.BlockSpec((1, Lb, D), lambda b_, l_: (b_, l_, 0)),
            pl.BlockSpec(memory_space=pltpu.SMEM),
            pl.BlockSpec((1, 2, D), lambda b_, l_: (b_, 0, 0)),
            pl.BlockSpec((D, D), lambda b_, l_: (0, 0)),
            pl.BlockSpec((D, D), lambda b_, l_: (0, 0)),
        ],
        out_specs=pl.BlockSpec((1, Lb, D), lambda b_, l_: (b_, l_, 0)),
        out_shape=jax.ShapeDtypeStruct((B, L, D), jnp.float32),
        compiler_params=pltpu.CompilerParams(
            dimension_semantics=("parallel", "arbitrary"),
            vmem_limit_bytes=56 * 1024 * 1024,
        ),
        name="dalm_pass2",
    )(x, memory_snapshot, raw, attn, pw, f1, f2)
    return y
